# Initial kernel scaffold; baseline (speedup 1.0000x reference)
import jax
import jax.numpy as jnp
from jax.experimental import pallas as pl


def kernel(instance_ids, table_instance, table_backgrounds):
    raise NotImplementedError("write your pallas kernel here")


# speedup vs baseline: 1.5324x; 1.5324x over previous
import functools

import jax
import jax.numpy as jnp
from jax import lax
from jax.experimental import pallas as pl
from jax.experimental.pallas import tpu as pltpu
from jax.experimental.pallas import tpu_sc as plsc

_D = 128
_B = 16384

_info = plsc.get_sparse_core_info()
_NC = _info.num_cores
_NS = _info.num_subcores
_NW = _NC * _NS
_BPW = _B // _NW
_CH = 128
_NCH = _BPW // _CH

_mesh = plsc.VectorSubcoreMesh(core_axis_name="c", subcore_axis_name="s")


@functools.partial(
    pl.kernel,
    mesh=_mesh,
    out_type=(
        jax.ShapeDtypeStruct((_B, _D), jnp.float32),
        jax.ShapeDtypeStruct((_B, _D), jnp.float32),
    ),
    scratch_types=[
        pltpu.VMEM((_NCH, _CH), jnp.int32),
        pltpu.VMEM((2, _CH, _D), jnp.float32),
        pltpu.SemaphoreType.DMA,
        pltpu.SemaphoreType.DMA,
        pltpu.SemaphoreType.DMA,
        pltpu.SemaphoreType.DMA,
    ],
)
def _dual_gather(ids_hbm, t1_hbm, t2_hbm, out1_hbm, out2_hbm,
                 idx_v, bufs, g0, g1, w0, w1):
    wid = lax.axis_index("s") * _NC + lax.axis_index("c")
    base = wid * _BPW
    pltpu.sync_copy(ids_hbm.at[pl.ds(wid * _NCH, _NCH)], idx_v)

    tables = (t1_hbm, t2_hbm)
    outs = (out1_hbm, out2_hbm)
    gsems = (g0, g1)
    wsems = (w0, w1)
    items = [(t, j) for t in (0, 1) for j in range(_NCH)]
    n = len(items)

    gh = [None] * n
    wh = [None] * n
    for i in range(2):
        t, j = items[i]
        gh[i] = pltpu.async_copy(
            tables[t].at[idx_v.at[j]], bufs.at[i % 2], gsems[i % 2])
    for i in range(n):
        t, j = items[i]
        b = i % 2
        gh[i].wait()
        wh[i] = pltpu.async_copy(
            bufs.at[b], outs[t].at[pl.ds(base + j * _CH, _CH)], wsems[b])
        if i + 2 < n:
            t2, j2 = items[i + 2]
            wh[i].wait()
            gh[i + 2] = pltpu.async_copy(
                tables[t2].at[idx_v.at[j2]], bufs.at[b], gsems[b])
    wh[n - 2].wait()
    wh[n - 1].wait()


def kernel(instance_ids, table_instance, table_backgrounds):
    ids2d = instance_ids.astype(jnp.int32).reshape(_B // _CH, _CH)
    out1, out2 = _dual_gather(ids2d, table_instance, table_backgrounds)
    return (out1, out2)


# speedup vs baseline: 1.5663x; 1.0221x over previous
import functools

import jax
import jax.numpy as jnp
from jax import lax
from jax.experimental import pallas as pl
from jax.experimental.pallas import tpu as pltpu
from jax.experimental.pallas import tpu_sc as plsc

_D = 128
_B = 16384

_info = plsc.get_sparse_core_info()
_NC = _info.num_cores
_NS = _info.num_subcores
_NW = _NC * _NS
_BPW = _B // _NW
_CH = 128
_NCH = _BPW // _CH

_mesh = plsc.VectorSubcoreMesh(core_axis_name="c", subcore_axis_name="s")


@functools.partial(
    pl.kernel,
    mesh=_mesh,
    out_type=(
        jax.ShapeDtypeStruct((_B, _D), jnp.float32),
        jax.ShapeDtypeStruct((_B, _D), jnp.float32),
    ),
    scratch_types=[
        pltpu.VMEM((_NCH, _CH), jnp.int32),
        pltpu.VMEM((4, _CH, _D), jnp.float32),
        pltpu.SemaphoreType.DMA,
        pltpu.SemaphoreType.DMA,
        pltpu.SemaphoreType.DMA,
        pltpu.SemaphoreType.DMA,
        pltpu.SemaphoreType.DMA,
        pltpu.SemaphoreType.DMA,
        pltpu.SemaphoreType.DMA,
        pltpu.SemaphoreType.DMA,
    ],
)
def _dual_gather(ids_hbm, t1_hbm, t2_hbm, out1_hbm, out2_hbm,
                 idx_v, bufs, g0, g1, g2, g3, w0, w1, w2, w3):
    wid = lax.axis_index("s") * _NC + lax.axis_index("c")
    base = wid * _BPW
    pltpu.sync_copy(ids_hbm.at[pl.ds(wid * _NCH, _NCH)], idx_v)

    tables = (t1_hbm, t2_hbm)
    outs = (out1_hbm, out2_hbm)
    gsems = (g0, g1, g2, g3)
    wsems = (w0, w1, w2, w3)
    items = [(t, j) for t in (0, 1) for j in range(_NCH)]
    n = len(items)
    nb = 4
    depth = nb - 1

    gh = [None] * n
    wh = [None] * n
    for i in range(depth):
        t, j = items[i]
        gh[i] = pltpu.async_copy(
            tables[t].at[idx_v.at[j]], bufs.at[i % nb], gsems[i % nb])
    for i in range(n):
        t, j = items[i]
        b = i % nb
        gh[i].wait()
        wh[i] = pltpu.async_copy(
            bufs.at[b], outs[t].at[pl.ds(base + j * _CH, _CH)], wsems[b])
        if i + depth < n:
            tn, jn = items[i + depth]
            bn = (i + depth) % nb
            if i > 0:
                wh[i - 1].wait()
            gh[i + depth] = pltpu.async_copy(
                tables[tn].at[idx_v.at[jn]], bufs.at[bn], gsems[bn])
    for i in range(n - nb, n):
        if i >= 0:
            wh[i].wait()


def kernel(instance_ids, table_instance, table_backgrounds):
    ids2d = instance_ids.astype(jnp.int32).reshape(_B // _CH, _CH)
    out1, out2 = _dual_gather(ids2d, table_instance, table_backgrounds)
    return (out1, out2)


# speedup vs baseline: 1.5704x; 1.0026x over previous
import functools

import jax
import jax.numpy as jnp
from jax import lax
from jax.experimental import pallas as pl
from jax.experimental.pallas import tpu as pltpu
from jax.experimental.pallas import tpu_sc as plsc

_D = 128
_B = 16384

_info = plsc.get_sparse_core_info()
_NC = _info.num_cores
_NS = _info.num_subcores
_NW = _NC * _NS
_BPW = _B // _NW
_CH = 64
_NCH = _BPW // _CH
_NB = 8

_mesh = plsc.VectorSubcoreMesh(core_axis_name="c", subcore_axis_name="s")


@functools.partial(
    pl.kernel,
    mesh=_mesh,
    out_type=(
        jax.ShapeDtypeStruct((_B, _D), jnp.float32),
        jax.ShapeDtypeStruct((_B, _D), jnp.float32),
    ),
    scratch_types=(
        [pltpu.VMEM((_NCH, _CH), jnp.int32),
         pltpu.VMEM((_NB, _CH, _D), jnp.float32)]
        + [pltpu.SemaphoreType.DMA] * (2 * _NB)
    ),
)
def _dual_gather(ids_hbm, t1_hbm, t2_hbm, out1_hbm, out2_hbm,
                 idx_v, bufs, *sems):
    gsems = sems[:_NB]
    wsems = sems[_NB:]
    wid = lax.axis_index("s") * _NC + lax.axis_index("c")
    base = wid * _BPW
    pltpu.sync_copy(ids_hbm.at[pl.ds(wid * _NCH, _NCH)], idx_v)

    tables = (t1_hbm, t2_hbm)
    outs = (out1_hbm, out2_hbm)
    items = [(t, j) for t in (0, 1) for j in range(_NCH)]
    n = len(items)
    depth = _NB - 1

    gh = [None] * n
    wh = [None] * n
    for i in range(depth):
        t, j = items[i]
        gh[i] = pltpu.async_copy(
            tables[t].at[idx_v.at[j]], bufs.at[i % _NB], gsems[i % _NB])
    for i in range(n):
        t, j = items[i]
        b = i % _NB
        gh[i].wait()
        wh[i] = pltpu.async_copy(
            bufs.at[b], outs[t].at[pl.ds(base + j * _CH, _CH)], wsems[b])
        if i + depth < n:
            tn, jn = items[i + depth]
            bn = (i + depth) % _NB
            if i > 0:
                wh[i - 1].wait()
            gh[i + depth] = pltpu.async_copy(
                tables[tn].at[idx_v.at[jn]], bufs.at[bn], gsems[bn])
    for i in range(max(0, n - _NB), n):
        if wh[i] is not None and i >= n - _NB:
            wh[i].wait()


def kernel(instance_ids, table_instance, table_backgrounds):
    ids2d = instance_ids.astype(jnp.int32).reshape(_B // _CH, _CH)
    out1, out2 = _dual_gather(ids2d, table_instance, table_backgrounds)
    return (out1, out2)
